# baseline (speedup 1.0000x reference)
import functools

import jax
import jax.numpy as jnp
from jax.experimental import pallas as pl
from jax.experimental.pallas import tpu as pltpu

N = 4096
INPUT_DIM = 256
DIM = 64
K = 1024
M_BOOK = 4
ROW_TILE = 128


def _bn_relu(h, g, b, eps=1e-5):
    mu = jnp.mean(h, axis=0, keepdims=True)
    c = h - mu
    var = jnp.mean(c * c, axis=0, keepdims=True)
    return jnp.maximum((h - mu) / jnp.sqrt(var + eps) * g + b, 0.0)


def _rowmin(s):
    w = s.shape[1]
    while w > 128:
        s = jnp.minimum(s[:, :w // 2], s[:, w // 2:w])
        w //= 2
    return jnp.min(s, axis=1, keepdims=True)


def _dotT(a, b):
    return jax.lax.dot_general(a, b, (((1,), (1,)), ((), ())),
                               preferred_element_type=jnp.float32)


def _lanesum64(s):
    acc = s[:, 0:8]
    for i in range(1, 8):
        acc = acc + s[:, 8 * i:8 * i + 8]
    w = 8
    while w > 1:
        acc = acc[:, :w // 2] + acc[:, w // 2:w]
        w //= 2
    return acc


def _fused_kernel(x_ref, w1_ref, b1_ref, g1_ref, be1_ref,
                  w2_ref, b2_ref, g2_ref, be2_ref,
                  w3_ref, b3_ref,
                  cb_ref, ee_ref,
                  dw1_ref, db1_ref, dg1_ref, dbe1_ref,
                  dw2_ref, db2_ref, dg2_ref, dbe2_ref,
                  dw3_ref, db3_ref,
                  xhat_ref, res_ref, ce_ref,
                  decin_ref):
    h = _bn_relu(_dotT(x_ref[:], w1_ref[:]) + b1_ref[:],
                 g1_ref[:], be1_ref[:])
    h = _bn_relu(_dotT(h, w2_ref[:]) + b2_ref[:], g2_ref[:], be2_ref[:])
    ze = _dotT(h, w3_ref[:]) + b3_ref[:]

    cb_parts = []
    for m in range(M_BOOK):
        cb = cb_ref[m]
        hi = cb.astype(jnp.bfloat16)
        r1 = cb - hi.astype(jnp.float32)
        mid = r1.astype(jnp.bfloat16)
        lo = (r1 - mid.astype(jnp.float32)).astype(jnp.bfloat16)
        cb_parts.append(jnp.concatenate([hi, mid, lo], axis=1))
    iota = jax.lax.broadcasted_iota(jnp.int32, (ROW_TILE, K), 1)
    for t in range(N // ROW_TILE):
        rows = slice(t * ROW_TILE, (t + 1) * ROW_TILE)
        r = ze[rows]
        zq = jnp.zeros_like(r)
        for m in range(M_BOOK):
            rr = _lanesum64(r * r)
            dist = rr + ee_ref[m] - 2.0 * _dotT(r, cb_ref[m])
            rowmin = _rowmin(dist)
            first = _rowmin(jnp.where(dist == rowmin, iota, K))
            onehot = (iota == first).astype(jnp.bfloat16)
            ce3 = jnp.dot(onehot, cb_parts[m],
                          preferred_element_type=jnp.float32)
            ce = (ce3[:, 0:DIM] + ce3[:, DIM:2 * DIM]) + ce3[:, 2 * DIM:]
            res_ref[m, rows, :] = r
            ce_ref[m, rows, :] = ce
            zq = zq + ce
            r = r - ce
        decin_ref[rows, :] = ze[rows] + (zq - ze[rows])

    h = _bn_relu(_dotT(decin_ref[:], dw1_ref[:]) + db1_ref[:],
                 dg1_ref[:], dbe1_ref[:])
    h = _bn_relu(_dotT(h, dw2_ref[:]) + db2_ref[:], dg2_ref[:], dbe2_ref[:])
    xhat_ref[:] = _dotT(h, dw3_ref[:]) + db3_ref[:]


@jax.jit
def kernel(x, enc_W1, enc_b1, bn1_g, bn1_b, enc_W2, enc_b2, bn2_g, bn2_b,
           enc_W3, enc_b3, codebooks, dec_W1, dec_b1, dbn1_g, dbn1_b,
           dec_W2, dec_b2, dbn2_g, dbn2_b, dec_W3, dec_b3):
    row2 = lambda v: v.reshape(1, -1)
    ee = jnp.sum(codebooks * codebooks, axis=2)[:, None, :]
    args = (
        x, enc_W1, row2(enc_b1), row2(bn1_g), row2(bn1_b),
        enc_W2, row2(enc_b2), row2(bn2_g), row2(bn2_b),
        enc_W3, row2(enc_b3),
        codebooks, ee,
        dec_W1, row2(dec_b1), row2(dbn1_g), row2(dbn1_b),
        dec_W2, row2(dec_b2), row2(dbn2_g), row2(dbn2_b),
        dec_W3, row2(dec_b3),
    )
    out_shapes = (
        jax.ShapeDtypeStruct((N, INPUT_DIM), jnp.float32),
        jax.ShapeDtypeStruct((M_BOOK, N, DIM), jnp.float32),
        jax.ShapeDtypeStruct((M_BOOK, N, DIM), jnp.float32),
    )
    x_hat, res_st, ce_st = pl.pallas_call(
        _fused_kernel,
        out_shape=out_shapes,
        scratch_shapes=[pltpu.VMEM((N, DIM), jnp.float32)],
    )(*args)
    return (x_hat, res_st, ce_st)


# speedup vs baseline: 1.2953x; 1.2953x over previous
import jax
import jax.numpy as jnp
from jax.experimental import pallas as pl
from jax.experimental.pallas import tpu as pltpu

N = 4096
INPUT_DIM = 256
DIM = 64
K = 1024
M_BOOK = 4
ROW_TILE = 256


def _bn_relu(h, g, b, eps=1e-5):
    mu = jnp.mean(h, axis=0, keepdims=True)
    c = h - mu
    var = jnp.mean(c * c, axis=0, keepdims=True)
    return jnp.maximum((h - mu) / jnp.sqrt(var + eps) * g + b, 0.0)


def _rowmin(s):
    w = s.shape[1]
    while w > 128:
        s = jnp.minimum(s[:, :w // 2], s[:, w // 2:w])
        w //= 2
    return jnp.min(s, axis=1, keepdims=True)


def _dotT(a, b):
    return jax.lax.dot_general(a, b, (((1,), (1,)), ((), ())),
                               preferred_element_type=jnp.float32)


def _lanesum64(s):
    acc = s[:, 0:8]
    for i in range(1, 8):
        acc = acc + s[:, 8 * i:8 * i + 8]
    w = 8
    while w > 1:
        acc = acc[:, :w // 2] + acc[:, w // 2:w]
        w //= 2
    return acc


def _fused_kernel(x_ref, w1_ref, b1_ref, g1_ref, be1_ref,
                  w2_ref, b2_ref, g2_ref, be2_ref,
                  w3_ref, b3_ref,
                  cb_ref, ee_ref,
                  dw1_ref, db1_ref, dg1_ref, dbe1_ref,
                  dw2_ref, db2_ref, dg2_ref, dbe2_ref,
                  dw3_ref, db3_ref,
                  xhat_ref, res_ref, ce_ref,
                  decin_ref):
    h = _bn_relu(_dotT(x_ref[:], w1_ref[:]) + b1_ref[:],
                 g1_ref[:], be1_ref[:])
    h = _bn_relu(_dotT(h, w2_ref[:]) + b2_ref[:], g2_ref[:], be2_ref[:])
    ze = _dotT(h, w3_ref[:]) + b3_ref[:]

    cb_parts = []
    for m in range(M_BOOK):
        cb = cb_ref[m]
        hi = cb.astype(jnp.bfloat16)
        r1 = cb - hi.astype(jnp.float32)
        mid = r1.astype(jnp.bfloat16)
        lo = (r1 - mid.astype(jnp.float32)).astype(jnp.bfloat16)
        cb_parts.append(jnp.concatenate([hi, mid, lo], axis=1))
    iota = jax.lax.broadcasted_iota(jnp.int32, (ROW_TILE, K), 1)
    for t in range(N // ROW_TILE):
        rows = slice(t * ROW_TILE, (t + 1) * ROW_TILE)
        r = ze[rows]
        zq = jnp.zeros_like(r)
        for m in range(M_BOOK):
            rr = _lanesum64(r * r)
            dist = rr + ee_ref[m] - 2.0 * _dotT(r, cb_ref[m])
            rowmin = _rowmin(dist)
            first = _rowmin(jnp.where(dist == rowmin, iota, K))
            onehot = (iota == first).astype(jnp.bfloat16)
            ce3 = jnp.dot(onehot, cb_parts[m],
                          preferred_element_type=jnp.float32)
            ce = (ce3[:, 0:DIM] + ce3[:, DIM:2 * DIM]) + ce3[:, 2 * DIM:]
            res_ref[m, rows, :] = r
            ce_ref[m, rows, :] = ce
            zq = zq + ce
            r = r - ce
        decin_ref[rows, :] = ze[rows] + (zq - ze[rows])

    h = _bn_relu(_dotT(decin_ref[:], dw1_ref[:]) + db1_ref[:],
                 dg1_ref[:], dbe1_ref[:])
    h = _bn_relu(_dotT(h, dw2_ref[:]) + db2_ref[:], dg2_ref[:], dbe2_ref[:])
    xhat_ref[:] = _dotT(h, dw3_ref[:]) + db3_ref[:]


@jax.jit
def kernel(x, enc_W1, enc_b1, bn1_g, bn1_b, enc_W2, enc_b2, bn2_g, bn2_b,
           enc_W3, enc_b3, codebooks, dec_W1, dec_b1, dbn1_g, dbn1_b,
           dec_W2, dec_b2, dbn2_g, dbn2_b, dec_W3, dec_b3):
    row2 = lambda v: v.reshape(1, -1)
    ee = jnp.sum(codebooks * codebooks, axis=2)[:, None, :]
    args = (
        x, enc_W1, row2(enc_b1), row2(bn1_g), row2(bn1_b),
        enc_W2, row2(enc_b2), row2(bn2_g), row2(bn2_b),
        enc_W3, row2(enc_b3),
        codebooks, ee,
        dec_W1, row2(dec_b1), row2(dbn1_g), row2(dbn1_b),
        dec_W2, row2(dec_b2), row2(dbn2_g), row2(dbn2_b),
        dec_W3, row2(dec_b3),
    )
    out_shapes = (
        jax.ShapeDtypeStruct((N, INPUT_DIM), jnp.float32),
        jax.ShapeDtypeStruct((M_BOOK, N, DIM), jnp.float32),
        jax.ShapeDtypeStruct((M_BOOK, N, DIM), jnp.float32),
    )
    x_hat, res_st, ce_st = pl.pallas_call(
        _fused_kernel,
        out_shape=out_shapes,
        scratch_shapes=[pltpu.VMEM((N, DIM), jnp.float32)],
    )(*args)
    return (x_hat, res_st, ce_st)
